# baseline (speedup 1.0000x reference)
import functools

import jax
import jax.numpy as jnp
import numpy as np
from jax import lax
from jax.experimental import pallas as pl
from jax.experimental.pallas import tpu as pltpu
from jax.experimental.pallas import tpu_sc as plsc

EMBED_SIZE = 768
D_TYPE = EMBED_SIZE // 4
MAX_SEQ = 24
BASE_GSD = 10.0

NBUF = 6
NSTR = 2
SROW = 18


def _sincos_1d(pos, dim):
    omega = 1.0 / (10000.0 ** (jnp.arange(dim // 2, dtype=jnp.float32) / (dim / 2.0)))
    out = pos.astype(jnp.float32)[:, None] * omega[None, :]
    return jnp.concatenate([jnp.sin(out), jnp.cos(out)], axis=-1)


def _month_table(dim):
    angles = jnp.arange(0, 13, dtype=jnp.float32) / (12.0 / (2.0 * np.pi))
    ang = jnp.stack([angles] * (dim // 2), axis=-1)
    return jnp.concatenate([jnp.sin(ang)[:-1], jnp.cos(ang)[:-1]], axis=-1)


def _emb_from_grid_1d(pos, dim):
    omega = 1.0 / (10000.0 ** (jnp.arange(dim // 2, dtype=jnp.float32) / (dim / 2.0)))
    flat = pos.reshape(pos.shape[0], -1)
    out = flat[..., None] * omega[None, None, :]
    return jnp.concatenate([jnp.sin(out), jnp.cos(out)], axis=-1)


def _spatial_table(grid_size, res, dim):
    coords = jnp.arange(grid_size, dtype=jnp.float32)
    gw, gh = jnp.meshgrid(coords, coords, indexing='xy')
    grid = jnp.stack([gw, gh], axis=0)
    grid = grid[None, :, :, :] * res[:, None, None, None]
    emb_h = _emb_from_grid_1d(grid[:, 0], dim // 2)
    emb_w = _emb_from_grid_1d(grid[:, 1], dim // 2)
    return jnp.concatenate([emb_h, emb_w], axis=-1)


_SC_TC_PER_W = 5


@jax.jit
def _sc_month_lookup(months16, mtflat, a36flat):
    info = plsc.get_sparse_core_info()
    nc = info.num_cores
    mesh = plsc.VectorSubcoreMesh(core_axis_name="c", subcore_axis_name="s")

    @functools.partial(
        pl.kernel,
        mesh=mesh,
        out_type=jax.ShapeDtypeStruct((4 * 36 * EMBED_SIZE,), jnp.float32),
        scratch_types=[
            pltpu.VMEM((16,), jnp.int32),
            pltpu.VMEM((16, EMBED_SIZE), jnp.float32),
            pltpu.VMEM((EMBED_SIZE,), jnp.float32),
            pltpu.SemaphoreType.DMA,
        ],
    )
    def k(months_hbm, mt_hbm, a36_hbm, out_hbm, idx_v, mo_v, row_v, sem):
        wid = lax.axis_index("s") * nc + lax.axis_index("c")
        b = wid // 8
        chunk = lax.rem(wid, 8)
        pltpu.sync_copy(months_hbm.at[b], idx_v)
        pltpu.async_copy(mt_hbm.at[idx_v], mo_v, sem).wait()
        for kk in range(_SC_TC_PER_W):
            tc = chunk * _SC_TC_PER_W + kk

            @pl.when(tc < 36)
            def _do(tc=tc):
                t = tc // 3
                src_off = pl.multiple_of(tc * EMBED_SIZE, EMBED_SIZE)
                pltpu.sync_copy(a36_hbm.at[pl.ds(src_off, EMBED_SIZE)], row_v)
                for j in range(12):
                    sl = row_v[pl.ds(384 + j * 16, 16)]
                    mo = mo_v[t, pl.ds(384 + j * 16, 16)]
                    row_v[pl.ds(384 + j * 16, 16)] = sl + mo
                dst_off = pl.multiple_of((b * 36 + tc) * EMBED_SIZE, EMBED_SIZE)
                pltpu.sync_copy(row_v, out_hbm.at[pl.ds(dst_off, EMBED_SIZE)])

    return k(months16, mtflat, a36flat)


def _add_kernel(ae_ref, s_ref, tok_hbm, out_hbm,
                in_buf, out_buf, sem_in, sem_out):
    i = pl.program_id(0)
    n = pl.num_programs(0)
    slot = jax.lax.rem(i, NBUF)

    @pl.when(i == 0)
    def _prologue():
        for j in range(NBUF - 1):
            for st in range(NSTR):
                pltpu.make_async_copy(tok_hbm.at[j, pl.ds(st * SROW, SROW)],
                                      in_buf.at[j, pl.ds(st * SROW, SROW)],
                                      sem_in.at[j, st]).start()

    nxt = i + NBUF - 1

    @pl.when(nxt < n)
    def _issue_in():
        nslot = jax.lax.rem(nxt, NBUF)
        for st in range(NSTR):
            pltpu.make_async_copy(tok_hbm.at[nxt, pl.ds(st * SROW, SROW)],
                                  in_buf.at[nslot, pl.ds(st * SROW, SROW)],
                                  sem_in.at[nslot, st]).start()

    for st in range(NSTR):
        pltpu.make_async_copy(tok_hbm.at[i, pl.ds(st * SROW, SROW)],
                              in_buf.at[slot, pl.ds(st * SROW, SROW)],
                              sem_in.at[slot, st]).wait()

    @pl.when(i >= NBUF)
    def _wait_out():
        for st in range(NSTR):
            pltpu.make_async_copy(out_buf.at[slot, pl.ds(st * SROW, SROW)],
                                  out_hbm.at[i, pl.ds(st * SROW, SROW)],
                                  sem_out.at[slot, st]).wait()

    bsel = i // 16
    ae = ae_ref[bsel]
    s = s_ref[i]
    out_buf[slot] = in_buf[slot] + ae[:, None, :] + s[None, :, :]

    for st in range(NSTR):
        pltpu.make_async_copy(out_buf.at[slot, pl.ds(st * SROW, SROW)],
                              out_hbm.at[i, pl.ds(st * SROW, SROW)],
                              sem_out.at[slot, st]).start()

    @pl.when(i == n - 1)
    def _drain():
        for j in range(NBUF):
            k = n - NBUF + j
            kslot = k % NBUF
            for st in range(NSTR):
                pltpu.make_async_copy(out_buf.at[kslot, pl.ds(st * SROW, SROW)],
                                      out_hbm.at[k, pl.ds(st * SROW, SROW)],
                                      sem_out.at[kslot, st]).wait()


@jax.jit
def _run(tokens4, ae_all, s_table):
    n, r, w, d = tokens4.shape
    return pl.pallas_call(
        _add_kernel,
        grid=(n,),
        in_specs=[
            pl.BlockSpec(memory_space=pltpu.MemorySpace.VMEM),
            pl.BlockSpec(memory_space=pltpu.MemorySpace.VMEM),
            pl.BlockSpec(memory_space=pltpu.MemorySpace.HBM),
        ],
        out_specs=pl.BlockSpec(memory_space=pltpu.MemorySpace.HBM),
        out_shape=jax.ShapeDtypeStruct(tokens4.shape, tokens4.dtype),
        scratch_shapes=[
            pltpu.VMEM((NBUF, r, w, d), jnp.float32),
            pltpu.VMEM((NBUF, r, w, d), jnp.float32),
            pltpu.SemaphoreType.DMA((NBUF, NSTR)),
            pltpu.SemaphoreType.DMA((NBUF, NSTR)),
        ],
    )(ae_all, s_table, tokens4)


def kernel(tokens, channel_embeddings, timestamps, patch_size, input_res):
    b, h, w, t, c, d = tokens.shape
    dt = d // 4

    pos = _sincos_1d(jnp.arange(MAX_SEQ), dt)[:t]
    a_table = jnp.concatenate(
        [jnp.broadcast_to(channel_embeddings[None, :, :], (t, c, dt)),
         jnp.broadcast_to(pos[:, None, :], (t, c, dt)),
         jnp.zeros((t, c, 2 * dt), dtype=jnp.float32)],
        axis=-1).reshape(t * c, d)

    gsd_ratio = (jnp.asarray(input_res).astype(jnp.float32)
                 * jnp.asarray(patch_size).astype(jnp.float32) / BASE_GSD)
    spatial = _spatial_table(h, jnp.ones((b,), dtype=jnp.float32) * gsd_ratio, dt)
    spatial = spatial.reshape(b, h, w, dt)
    s_table = jnp.concatenate(
        [jnp.zeros((b, h, w, 3 * dt), dtype=jnp.float32), spatial],
        axis=-1).reshape(b * h, w, d)

    mtable = jnp.concatenate(
        [jnp.zeros((12, 2 * dt), dtype=jnp.float32), _month_table(dt),
         jnp.zeros((12, dt), dtype=jnp.float32)], axis=-1)

    months = timestamps[:, 1, :].astype(jnp.int32)
    months16 = jnp.zeros((b, 16), jnp.int32).at[:, :t].set(months)

    ae_all = _sc_month_lookup(
        months16, mtable, a_table.reshape(-1)
    ).reshape(b, t * c, d)

    tokens4 = jnp.transpose(tokens, (0, 1, 3, 4, 2, 5)).reshape(b * h, t * c, w, d)
    out = _run(tokens4, ae_all, s_table)
    return jnp.transpose(out.reshape(b, h, t, c, w, d), (0, 1, 4, 2, 3, 5))


# speedup vs baseline: 1.1094x; 1.1094x over previous
import functools

import jax
import jax.numpy as jnp
import numpy as np
from jax import lax
from jax.experimental import pallas as pl
from jax.experimental.pallas import tpu as pltpu
from jax.experimental.pallas import tpu_sc as plsc

EMBED_SIZE = 768
D_TYPE = EMBED_SIZE // 4
MAX_SEQ = 24
BASE_GSD = 10.0

NBUF = 6
NSTR = 2
SROW = 18


def _sincos_1d(pos, dim):
    omega = 1.0 / (10000.0 ** (jnp.arange(dim // 2, dtype=jnp.float32) / (dim / 2.0)))
    out = pos.astype(jnp.float32)[:, None] * omega[None, :]
    return jnp.concatenate([jnp.sin(out), jnp.cos(out)], axis=-1)


def _month_table(dim):
    angles = jnp.arange(0, 13, dtype=jnp.float32) / (12.0 / (2.0 * np.pi))
    ang = jnp.stack([angles] * (dim // 2), axis=-1)
    return jnp.concatenate([jnp.sin(ang)[:-1], jnp.cos(ang)[:-1]], axis=-1)


def _emb_from_grid_1d(pos, dim):
    omega = 1.0 / (10000.0 ** (jnp.arange(dim // 2, dtype=jnp.float32) / (dim / 2.0)))
    flat = pos.reshape(pos.shape[0], -1)
    out = flat[..., None] * omega[None, None, :]
    return jnp.concatenate([jnp.sin(out), jnp.cos(out)], axis=-1)


def _spatial_table(grid_size, res, dim):
    coords = jnp.arange(grid_size, dtype=jnp.float32)
    gw, gh = jnp.meshgrid(coords, coords, indexing='xy')
    grid = jnp.stack([gw, gh], axis=0)
    grid = grid[None, :, :, :] * res[:, None, None, None]
    emb_h = _emb_from_grid_1d(grid[:, 0], dim // 2)
    emb_w = _emb_from_grid_1d(grid[:, 1], dim // 2)
    return jnp.concatenate([emb_h, emb_w], axis=-1)


@jax.jit
def _sc_month_lookup(months16, mtflat):
    info = plsc.get_sparse_core_info()
    nc = info.num_cores
    mesh = plsc.VectorSubcoreMesh(core_axis_name="c", subcore_axis_name="s")

    @functools.partial(
        pl.kernel,
        mesh=mesh,
        out_type=jax.ShapeDtypeStruct((4, 16, EMBED_SIZE), jnp.float32),
        scratch_types=[
            pltpu.VMEM((16,), jnp.int32),
            pltpu.VMEM((16, EMBED_SIZE), jnp.float32),
            pltpu.SemaphoreType.DMA,
        ],
    )
    def k(months_hbm, mt_hbm, out_hbm, idx_v, mo_v, sem):
        wid = lax.axis_index("s") * nc + lax.axis_index("c")

        @pl.when(wid < 4)
        def _do():
            pltpu.sync_copy(months_hbm.at[wid], idx_v)
            pltpu.async_copy(mt_hbm.at[idx_v], mo_v, sem).wait()
            pltpu.sync_copy(mo_v, out_hbm.at[wid])

    return k(months16, mtflat)


def _add_kernel(a_ref, mo_ref, s_ref, tok_hbm, out_hbm,
                in_buf, out_buf, ae_buf, sem_in, sem_out):
    i = pl.program_id(0)
    n = pl.num_programs(0)
    slot = jax.lax.rem(i, NBUF)

    @pl.when(i == 0)
    def _prologue():
        r36 = jax.lax.broadcasted_iota(jnp.int32, (36, 12), 0) // 3
        t36 = jax.lax.broadcasted_iota(jnp.int32, (36, 12), 1)
        rep = (r36 == t36).astype(jnp.float32)
        for bb in range(4):
            mo36 = jnp.dot(rep, mo_ref[bb, :12, :],
                           preferred_element_type=jnp.float32)
            ae_buf[bb] = a_ref[...] + mo36
        for j in range(NBUF - 1):
            for st in range(NSTR):
                pltpu.make_async_copy(tok_hbm.at[j, pl.ds(st * SROW, SROW)],
                                      in_buf.at[j, pl.ds(st * SROW, SROW)],
                                      sem_in.at[j, st]).start()

    nxt = i + NBUF - 1

    @pl.when(nxt < n)
    def _issue_in():
        nslot = jax.lax.rem(nxt, NBUF)
        for st in range(NSTR):
            pltpu.make_async_copy(tok_hbm.at[nxt, pl.ds(st * SROW, SROW)],
                                  in_buf.at[nslot, pl.ds(st * SROW, SROW)],
                                  sem_in.at[nslot, st]).start()

    for st in range(NSTR):
        pltpu.make_async_copy(tok_hbm.at[i, pl.ds(st * SROW, SROW)],
                              in_buf.at[slot, pl.ds(st * SROW, SROW)],
                              sem_in.at[slot, st]).wait()

    @pl.when(i >= NBUF)
    def _wait_out():
        for st in range(NSTR):
            pltpu.make_async_copy(out_buf.at[slot, pl.ds(st * SROW, SROW)],
                                  out_hbm.at[i, pl.ds(st * SROW, SROW)],
                                  sem_out.at[slot, st]).wait()

    bsel = i // 16
    ae = ae_buf[bsel]
    s = s_ref[i]
    out_buf[slot] = in_buf[slot] + ae[:, None, :] + s[None, :, :]

    for st in range(NSTR):
        pltpu.make_async_copy(out_buf.at[slot, pl.ds(st * SROW, SROW)],
                              out_hbm.at[i, pl.ds(st * SROW, SROW)],
                              sem_out.at[slot, st]).start()

    @pl.when(i == n - 1)
    def _drain():
        for j in range(NBUF):
            k = n - NBUF + j
            kslot = k % NBUF
            for st in range(NSTR):
                pltpu.make_async_copy(out_buf.at[kslot, pl.ds(st * SROW, SROW)],
                                      out_hbm.at[k, pl.ds(st * SROW, SROW)],
                                      sem_out.at[kslot, st]).wait()


@jax.jit
def _run(tokens4, a_table, mo_all, s_table):
    n, r, w, d = tokens4.shape
    return pl.pallas_call(
        _add_kernel,
        grid=(n,),
        in_specs=[
            pl.BlockSpec(memory_space=pltpu.MemorySpace.VMEM),
            pl.BlockSpec(memory_space=pltpu.MemorySpace.VMEM),
            pl.BlockSpec(memory_space=pltpu.MemorySpace.VMEM),
            pl.BlockSpec(memory_space=pltpu.MemorySpace.HBM),
        ],
        out_specs=pl.BlockSpec(memory_space=pltpu.MemorySpace.HBM),
        out_shape=jax.ShapeDtypeStruct(tokens4.shape, tokens4.dtype),
        scratch_shapes=[
            pltpu.VMEM((NBUF, r, w, d), jnp.float32),
            pltpu.VMEM((NBUF, r, w, d), jnp.float32),
            pltpu.VMEM((4, r, d), jnp.float32),
            pltpu.SemaphoreType.DMA((NBUF, NSTR)),
            pltpu.SemaphoreType.DMA((NBUF, NSTR)),
        ],
    )(a_table, mo_all, s_table, tokens4)


def kernel(tokens, channel_embeddings, timestamps, patch_size, input_res):
    b, h, w, t, c, d = tokens.shape
    dt = d // 4

    pos = _sincos_1d(jnp.arange(MAX_SEQ), dt)[:t]
    a_table = jnp.concatenate(
        [jnp.broadcast_to(channel_embeddings[None, :, :], (t, c, dt)),
         jnp.broadcast_to(pos[:, None, :], (t, c, dt)),
         jnp.zeros((t, c, 2 * dt), dtype=jnp.float32)],
        axis=-1).reshape(t * c, d)

    gsd_ratio = (jnp.asarray(input_res).astype(jnp.float32)
                 * jnp.asarray(patch_size).astype(jnp.float32) / BASE_GSD)
    spatial = _spatial_table(h, jnp.ones((b,), dtype=jnp.float32) * gsd_ratio, dt)
    spatial = spatial.reshape(b, h, w, dt)
    s_table = jnp.concatenate(
        [jnp.zeros((b, h, w, 3 * dt), dtype=jnp.float32), spatial],
        axis=-1).reshape(b * h, w, d)

    mtable = jnp.concatenate(
        [jnp.zeros((12, 2 * dt), dtype=jnp.float32), _month_table(dt),
         jnp.zeros((12, dt), dtype=jnp.float32)], axis=-1)

    months = timestamps[:, 1, :].astype(jnp.int32)
    months16 = jnp.zeros((b, 16), jnp.int32).at[:, :t].set(months)

    mo_all = _sc_month_lookup(months16, mtable)

    tokens4 = jnp.transpose(tokens, (0, 1, 3, 4, 2, 5)).reshape(b * h, t * c, w, d)
    out = _run(tokens4, a_table, mo_all, s_table)
    return jnp.transpose(out.reshape(b, h, t, c, w, d), (0, 1, 4, 2, 3, 5))


# speedup vs baseline: 1.1355x; 1.0236x over previous
import functools

import jax
import jax.numpy as jnp
import numpy as np
from jax import lax
from jax.experimental import pallas as pl
from jax.experimental.pallas import tpu as pltpu
from jax.experimental.pallas import tpu_sc as plsc

EMBED_SIZE = 768
D_TYPE = EMBED_SIZE // 4
MAX_SEQ = 24
BASE_GSD = 10.0

NBUF = 6
NSTR = 2
SROW = 18


def _sincos_1d(pos, dim):
    omega = 1.0 / (10000.0 ** (jnp.arange(dim // 2, dtype=jnp.float32) / (dim / 2.0)))
    out = pos.astype(jnp.float32)[:, None] * omega[None, :]
    return jnp.concatenate([jnp.sin(out), jnp.cos(out)], axis=-1)


def _month_table(dim):
    angles = jnp.arange(0, 13, dtype=jnp.float32) / (12.0 / (2.0 * np.pi))
    ang = jnp.stack([angles] * (dim // 2), axis=-1)
    return jnp.concatenate([jnp.sin(ang)[:-1], jnp.cos(ang)[:-1]], axis=-1)


def _emb_from_grid_1d(pos, dim):
    omega = 1.0 / (10000.0 ** (jnp.arange(dim // 2, dtype=jnp.float32) / (dim / 2.0)))
    flat = pos.reshape(pos.shape[0], -1)
    out = flat[..., None] * omega[None, None, :]
    return jnp.concatenate([jnp.sin(out), jnp.cos(out)], axis=-1)


def _spatial_table(grid_size, res, dim):
    coords = jnp.arange(grid_size, dtype=jnp.float32)
    gw, gh = jnp.meshgrid(coords, coords, indexing='xy')
    grid = jnp.stack([gw, gh], axis=0)
    grid = grid[None, :, :, :] * res[:, None, None, None]
    emb_h = _emb_from_grid_1d(grid[:, 0], dim // 2)
    emb_w = _emb_from_grid_1d(grid[:, 1], dim // 2)
    return jnp.concatenate([emb_h, emb_w], axis=-1)


@jax.jit
def _sc_month_lookup(months16, mtflat):
    info = plsc.get_sparse_core_info()
    nc = info.num_cores
    mesh = plsc.VectorSubcoreMesh(core_axis_name="c", subcore_axis_name="s")

    @functools.partial(
        pl.kernel,
        mesh=mesh,
        out_type=jax.ShapeDtypeStruct((4, 16, EMBED_SIZE), jnp.float32),
        scratch_types=[
            pltpu.VMEM((16,), jnp.int32),
            pltpu.VMEM((16, EMBED_SIZE), jnp.float32),
            pltpu.SemaphoreType.DMA,
        ],
    )
    def k(months_hbm, mt_hbm, out_hbm, idx_v, mo_v, sem):
        wid = lax.axis_index("s") * nc + lax.axis_index("c")

        @pl.when(wid < 4)
        def _do():
            pltpu.sync_copy(months_hbm.at[wid], idx_v)
            pltpu.async_copy(mt_hbm.at[idx_v], mo_v, sem).wait()
            pltpu.sync_copy(mo_v, out_hbm.at[wid])

    return k(months16, mtflat)


def _add_kernel(a_ref, mo_ref, s_ref, tok_hbm, out_hbm,
                in_buf, out_buf, ae_buf, sem_in, sem_out):
    i = pl.program_id(0)
    n = pl.num_programs(0)
    slot = jax.lax.rem(i, NBUF)

    @pl.when(i == 0)
    def _prologue():
        for j in range(NBUF - 1):
            for st in range(NSTR):
                pltpu.make_async_copy(tok_hbm.at[j, pl.ds(st * SROW, SROW)],
                                      in_buf.at[j, pl.ds(st * SROW, SROW)],
                                      sem_in.at[j, st]).start()
        r36 = jax.lax.broadcasted_iota(jnp.int32, (36, 12), 0) // 3
        t36 = jax.lax.broadcasted_iota(jnp.int32, (36, 12), 1)
        rep = (r36 == t36).astype(jnp.float32)
        for bb in range(4):
            mo36 = jnp.dot(rep, mo_ref[bb, :12, :],
                           preferred_element_type=jnp.float32)
            ae_buf[bb] = a_ref[...] + mo36

    nxt = i + NBUF - 1

    @pl.when(nxt < n)
    def _issue_in():
        nslot = jax.lax.rem(nxt, NBUF)
        for st in range(NSTR):
            pltpu.make_async_copy(tok_hbm.at[nxt, pl.ds(st * SROW, SROW)],
                                  in_buf.at[nslot, pl.ds(st * SROW, SROW)],
                                  sem_in.at[nslot, st]).start()

    for st in range(NSTR):
        pltpu.make_async_copy(tok_hbm.at[i, pl.ds(st * SROW, SROW)],
                              in_buf.at[slot, pl.ds(st * SROW, SROW)],
                              sem_in.at[slot, st]).wait()

    @pl.when(i >= NBUF)
    def _wait_out():
        for st in range(NSTR):
            pltpu.make_async_copy(out_buf.at[slot, pl.ds(st * SROW, SROW)],
                                  out_hbm.at[i, pl.ds(st * SROW, SROW)],
                                  sem_out.at[slot, st]).wait()

    bsel = i // 16
    ae = ae_buf[bsel]
    s = s_ref[jax.lax.rem(i, 16)]
    out_buf[slot] = in_buf[slot] + ae[:, None, :] + s[None, :, :]

    for st in range(NSTR):
        pltpu.make_async_copy(out_buf.at[slot, pl.ds(st * SROW, SROW)],
                              out_hbm.at[i, pl.ds(st * SROW, SROW)],
                              sem_out.at[slot, st]).start()

    @pl.when(i == n - 1)
    def _drain():
        for j in range(NBUF):
            k = n - NBUF + j
            kslot = k % NBUF
            for st in range(NSTR):
                pltpu.make_async_copy(out_buf.at[kslot, pl.ds(st * SROW, SROW)],
                                      out_hbm.at[k, pl.ds(st * SROW, SROW)],
                                      sem_out.at[kslot, st]).wait()


@jax.jit
def _run(tokens4, a_table, mo_all, s_table):
    n, r, w, d = tokens4.shape
    return pl.pallas_call(
        _add_kernel,
        grid=(n,),
        in_specs=[
            pl.BlockSpec(memory_space=pltpu.MemorySpace.VMEM),
            pl.BlockSpec(memory_space=pltpu.MemorySpace.VMEM),
            pl.BlockSpec(memory_space=pltpu.MemorySpace.VMEM),
            pl.BlockSpec(memory_space=pltpu.MemorySpace.HBM),
        ],
        out_specs=pl.BlockSpec(memory_space=pltpu.MemorySpace.HBM),
        out_shape=jax.ShapeDtypeStruct(tokens4.shape, tokens4.dtype),
        scratch_shapes=[
            pltpu.VMEM((NBUF, r, w, d), jnp.float32),
            pltpu.VMEM((NBUF, r, w, d), jnp.float32),
            pltpu.VMEM((4, r, d), jnp.float32),
            pltpu.SemaphoreType.DMA((NBUF, NSTR)),
            pltpu.SemaphoreType.DMA((NBUF, NSTR)),
        ],
    )(a_table, mo_all, s_table, tokens4)


def kernel(tokens, channel_embeddings, timestamps, patch_size, input_res):
    b, h, w, t, c, d = tokens.shape
    dt = d // 4

    pos = _sincos_1d(jnp.arange(MAX_SEQ), dt)[:t]
    a_table = jnp.concatenate(
        [jnp.broadcast_to(channel_embeddings[None, :, :], (t, c, dt)),
         jnp.broadcast_to(pos[:, None, :], (t, c, dt)),
         jnp.zeros((t, c, 2 * dt), dtype=jnp.float32)],
        axis=-1).reshape(t * c, d)

    gsd_ratio = (jnp.asarray(input_res).astype(jnp.float32)
                 * jnp.asarray(patch_size).astype(jnp.float32) / BASE_GSD)
    spatial = _spatial_table(h, jnp.ones((1,), dtype=jnp.float32) * gsd_ratio, dt)
    spatial = spatial.reshape(h, w, dt)
    s_table = jnp.concatenate(
        [jnp.zeros((h, w, 3 * dt), dtype=jnp.float32), spatial],
        axis=-1)

    mtable = jnp.concatenate(
        [jnp.zeros((12, 2 * dt), dtype=jnp.float32), _month_table(dt),
         jnp.zeros((12, dt), dtype=jnp.float32)], axis=-1)

    months = timestamps[:, 1, :].astype(jnp.int32)
    months16 = jnp.zeros((b, 16), jnp.int32).at[:, :t].set(months)

    mo_all = _sc_month_lookup(months16, mtable)

    tokens4 = jnp.transpose(tokens, (0, 1, 3, 4, 2, 5)).reshape(b * h, t * c, w, d)
    out = _run(tokens4, a_table, mo_all, s_table)
    return jnp.transpose(out.reshape(b, h, t, c, w, d), (0, 1, 4, 2, 3, 5))
